# Initial kernel scaffold; baseline (speedup 1.0000x reference)
import jax
import jax.numpy as jnp
from jax.experimental import pallas as pl


def kernel(x, gate_W, expert_W, expert_b):
    raise NotImplementedError("write your pallas kernel here")


# speedup vs baseline: 2.4531x; 2.4531x over previous
import functools

import jax
import jax.numpy as jnp
from jax.experimental import pallas as pl
from jax.experimental.pallas import tpu as pltpu

N_TOKENS = 2048
D_IN = 1024
D_OUT = 1024
N_EXP = 8
BALANCE_W = 0.01


def _moe_kernel(x_ref, gw_ref, ew_ref, eb_ref, y_ref, loss_ref, scores_ref):
    e = pl.program_id(0)

    @pl.when(e == 0)
    def _init():
        xf = x_ref[...]
        logits = jax.lax.dot_general(
            xf, gw_ref[...], (((1,), (1,)), ((), ())),
            preferred_element_type=jnp.float32)
        idx = jax.lax.broadcasted_iota(jnp.int32, logits.shape, 1)
        big = jnp.float32(3.4e38)
        m1 = jnp.max(logits, axis=1, keepdims=True)
        i1 = jnp.min(jnp.where(logits == m1, idx, N_EXP), axis=1,
                     keepdims=True)
        masked = jnp.where(idx == i1, -big, logits)
        m2 = jnp.max(masked, axis=1, keepdims=True)
        i2 = jnp.min(jnp.where(masked == m2, idx, N_EXP), axis=1,
                     keepdims=True)
        s2 = 1.0 / (1.0 + jnp.exp(m1 - m2))
        s1 = 1.0 - s2
        scores_ref[...] = jnp.where(
            idx == i1, s1, jnp.where(idx == i2, s2, 0.0))

    sc = scores_ref[...]
    lane = jax.lax.broadcasted_iota(jnp.int32, sc.shape, 1)
    s_col = jnp.sum(jnp.where(lane == e, sc, 0.0), axis=1,
                    keepdims=True)
    xw = jax.lax.dot_general(
        x_ref[...], ew_ref[0], (((1,), (1,)), ((), ())),
        preferred_element_type=jnp.float32)
    contrib = s_col * xw + s_col * eb_ref[0]

    @pl.when(e == 0)
    def _set():
        y_ref[...] = contrib

    @pl.when(e > 0)
    def _acc():
        y_ref[...] += contrib

    @pl.when(e == N_EXP - 1)
    def _loss():
        importance = jnp.sum(sc, axis=0)
        load = jnp.sum((sc > 0).astype(jnp.float32), axis=0)

        def cv_sq(v):
            mean = jnp.mean(v)
            var = jnp.sum((v - mean) ** 2) / (N_EXP - 1)
            return var / (mean * mean + 1e-10)

        loss = BALANCE_W * (cv_sq(importance) + cv_sq(load))
        loss_ref[...] = jnp.reshape(loss, (1, 1))


@functools.partial(jax.jit)
def _moe(xf, gate_W, expert_W, expert_b):
    y, loss = pl.pallas_call(
        _moe_kernel,
        grid=(N_EXP,),
        in_specs=[
            pl.BlockSpec((N_TOKENS, D_IN), lambda e: (0, 0)),
            pl.BlockSpec((N_EXP, D_IN), lambda e: (0, 0)),
            pl.BlockSpec((1, D_OUT, D_IN), lambda e: (e, 0, 0)),
            pl.BlockSpec((1, 1, D_OUT), lambda e: (e, 0, 0)),
        ],
        out_specs=[
            pl.BlockSpec((N_TOKENS, D_OUT), lambda e: (0, 0)),
            pl.BlockSpec((1, 1), lambda e: (0, 0)),
        ],
        out_shape=[
            jax.ShapeDtypeStruct((N_TOKENS, D_OUT), jnp.float32),
            jax.ShapeDtypeStruct((1, 1), jnp.float32),
        ],
        scratch_shapes=[pltpu.VMEM((N_TOKENS, N_EXP), jnp.float32)],
    )(xf, gate_W, expert_W, expert_b.reshape(N_EXP, 1, D_OUT))
    return y, loss


def kernel(x, gate_W, expert_W, expert_b):
    orig_shape = x.shape[:-1]
    xf = x.reshape(-1, D_IN)
    y, loss = _moe(xf, gate_W, expert_W, expert_b)
    return y.reshape(orig_shape + (D_OUT,)), loss[0, 0]


# speedup vs baseline: 2.5476x; 1.0386x over previous
import functools

import jax
import jax.numpy as jnp
from jax.experimental import pallas as pl
from jax.experimental.pallas import tpu as pltpu

N_TOKENS = 2048
D_IN = 1024
D_OUT = 1024
N_EXP = 8
BALANCE_W = 0.01


def _moe_kernel(x_ref, gw_ref, ew_ref, eb_ref, y_ref, loss_ref,
                scores_ref, xb_ref):
    e = pl.program_id(0)

    @pl.when(e == 0)
    def _init():
        xf = x_ref[...]
        xb_ref[...] = xf.astype(jnp.bfloat16)
        logits = jax.lax.dot_general(
            xf, gw_ref[...], (((1,), (1,)), ((), ())),
            preferred_element_type=jnp.float32)
        idx = jax.lax.broadcasted_iota(jnp.int32, logits.shape, 1)
        big = jnp.float32(3.4e38)
        m1 = jnp.max(logits, axis=1, keepdims=True)
        i1 = jnp.min(jnp.where(logits == m1, idx, N_EXP), axis=1,
                     keepdims=True)
        masked = jnp.where(idx == i1, -big, logits)
        m2 = jnp.max(masked, axis=1, keepdims=True)
        i2 = jnp.min(jnp.where(masked == m2, idx, N_EXP), axis=1,
                     keepdims=True)
        s2 = 1.0 / (1.0 + jnp.exp(m1 - m2))
        s1 = 1.0 - s2
        scores_ref[...] = jnp.where(
            idx == i1, s1, jnp.where(idx == i2, s2, 0.0))

    sc = scores_ref[...]
    lane = jax.lax.broadcasted_iota(jnp.int32, sc.shape, 1)
    s_col = jnp.sum(jnp.where(lane == e, sc, 0.0), axis=1,
                    keepdims=True)
    xw = jax.lax.dot_general(
        xb_ref[...], ew_ref[0].astype(jnp.bfloat16), (((1,), (1,)), ((), ())),
        preferred_element_type=jnp.float32)
    contrib = s_col * xw

    @pl.when(e == 0)
    def _set():
        y_ref[...] = contrib

    @pl.when(e > 0)
    def _acc():
        y_ref[...] += contrib

    @pl.when(e == N_EXP - 1)
    def _fini():
        y_ref[...] += jax.lax.dot_general(
            sc, eb_ref[...], (((1,), (0,)), ((), ())),
            preferred_element_type=jnp.float32)

        importance = jnp.sum(sc, axis=0)
        load = jnp.sum((sc > 0).astype(jnp.float32), axis=0)

        def cv_sq(v):
            mean = jnp.mean(v)
            var = jnp.sum((v - mean) ** 2) / (N_EXP - 1)
            return var / (mean * mean + 1e-10)

        loss = BALANCE_W * (cv_sq(importance) + cv_sq(load))
        loss_ref[...] = jnp.reshape(loss, (1, 1))


@functools.partial(jax.jit)
def _moe(xf, gate_W, expert_W, expert_b):
    y, loss = pl.pallas_call(
        _moe_kernel,
        grid=(N_EXP,),
        in_specs=[
            pl.BlockSpec((N_TOKENS, D_IN), lambda e: (0, 0)),
            pl.BlockSpec((N_EXP, D_IN), lambda e: (0, 0)),
            pl.BlockSpec((1, D_OUT, D_IN), lambda e: (e, 0, 0)),
            pl.BlockSpec((N_EXP, D_OUT), lambda e: (0, 0)),
        ],
        out_specs=[
            pl.BlockSpec((N_TOKENS, D_OUT), lambda e: (0, 0)),
            pl.BlockSpec((1, 1), lambda e: (0, 0)),
        ],
        out_shape=[
            jax.ShapeDtypeStruct((N_TOKENS, D_OUT), jnp.float32),
            jax.ShapeDtypeStruct((1, 1), jnp.float32),
        ],
        scratch_shapes=[
            pltpu.VMEM((N_TOKENS, N_EXP), jnp.float32),
            pltpu.VMEM((N_TOKENS, D_IN), jnp.bfloat16),
        ],
    )(xf, gate_W, expert_W, expert_b)
    return y, loss


def kernel(x, gate_W, expert_W, expert_b):
    orig_shape = x.shape[:-1]
    xf = x.reshape(-1, D_IN)
    y, loss = _moe(xf, gate_W, expert_W, expert_b)
    return y.reshape(orig_shape + (D_OUT,)), loss[0, 0]
